# baseline (speedup 1.0000x reference)
import functools

import jax
import jax.numpy as jnp
from jax import lax
from jax.experimental import pallas as pl
from jax.experimental.pallas import tpu as pltpu
from jax.experimental.pallas import tpu_sc as plsc


def kernel(indices, tables):
    B, F = indices.shape
    _, V, D = tables.shape

    info = plsc.get_sparse_core_info()
    NC, NS, L = info.num_cores, info.num_subcores, info.num_lanes
    NW = NC * NS
    HD = D // 2
    UNITS = F * 2
    VC = 2048
    VSH = 11
    TAILV = (V // 128) * 128
    NFULL = TAILV // VC
    VLAST = TAILV - NFULL * VC
    NCH = NFULL + 2
    NBUF = 4
    NGRP = B // L

    idx_t = jnp.pad(indices.T, ((0, NW - F), (0, 0))).reshape(
        NW, B // 128, 128)
    tab2 = jnp.swapaxes(tables, 1, 2).reshape(F * D, V)
    tail = jnp.pad(
        jnp.swapaxes(tables[:, TAILV:, :], 1, 2).reshape(F * D, V - TAILV),
        ((0, 0), (0, 128 - (V - TAILV))),
    )

    mesh = plsc.VectorSubcoreMesh(core_axis_name="c", subcore_axis_name="s")

    @functools.partial(
        pl.kernel,
        mesh=mesh,
        compiler_params=pltpu.CompilerParams(needs_layout_passes=False),
        out_type=jax.ShapeDtypeStruct((F, D, B), jnp.float32),
        scratch_types=[
            pltpu.VMEM((B // 128, 128), jnp.int32),
            pltpu.VMEM((4, HD, VC), jnp.float32),
            pltpu.VMEM((HD, 128), jnp.float32),
            pltpu.VMEM((HD, B), jnp.float32),
            pltpu.VMEM((B + L,), jnp.int32),
            pltpu.VMEM((B + L,), jnp.int32),
            pltpu.VMEM(((NCH + 1) * L,), jnp.int32),
            pltpu.SMEM((NCH + 1,), jnp.int32),
            pltpu.SemaphoreType.DMA,
            pltpu.SemaphoreType.DMA,
            pltpu.SemaphoreType.DMA,
            pltpu.SemaphoreType.DMA,
            pltpu.SemaphoreType.DMA,
        ],
    )
    def emb(idx_hbm, tab_hbm, tail_hbm, out_hbm, idxblk, slab, tailslab,
            outs, blist, vlist, cur, pref, sem0, sem1, sem2, sem3, semf):
        sems = (sem0, sem1, sem2, sem3)
        wid = lax.axis_index("s") * NC + lax.axis_index("c")
        lanes = lax.iota(jnp.int32, L)
        ones = jnp.ones((L,), jnp.int32)
        zeros16 = jnp.zeros((L,), jnp.int32)

        def unit_body(u):
            f = lax.rem(u, F)
            h = lax.div(u, F)
            row0 = pl.multiple_of(f * D + HD * h, 8)

            def fire(c, buf, sem, width):
                return pltpu.async_copy(
                    tab_hbm.at[pl.ds(row0, HD),
                               pl.ds(pl.multiple_of(c * VC, 128), width)],
                    slab.at[buf, :, pl.ds(0, width)],
                    sem,
                )

            for b in range(NBUF):
                fire(b, b, sems[b], VC)
            pltpu.sync_copy(idx_hbm.at[f], idxblk)
            pltpu.sync_copy(tail_hbm.at[pl.ds(row0, HD)], tailslab)

            def zero(i, carry):
                cur[pl.ds(pl.multiple_of(i * L, L), L)] = zeros16
                return carry

            lax.fori_loop(0, NCH + 1, zero, 0, unroll=False)

            def chunk_of(iv):
                return jnp.where(
                    iv >= TAILV, NCH - 1, lax.shift_right_logical(iv, VSH)
                )

            def hist(g, carry):
                iv = idxblk[lax.div(g, 8),
                            pl.ds(pl.multiple_of(lax.rem(g, 8) * L, L), L)]
                c = chunk_of(iv)
                plsc.addupdate_scatter(cur, [c * L + lanes], ones)
                return carry

            lax.fori_loop(0, NGRP, hist, 0, unroll=False)

            def prefix(c, carry):
                pref[c] = carry
                sl = pl.ds(pl.multiple_of(c * L, L), L)
                grp = cur[sl]
                inc = plsc.cumsum(grp)
                cur[sl] = inc - grp + carry
                return carry + jnp.sum(grp)

            total = lax.fori_loop(0, NCH, prefix, 0, unroll=False)
            pref[NCH] = total

            def append(g, carry):
                iv = idxblk[lax.div(g, 8),
                            pl.ds(pl.multiple_of(lax.rem(g, 8) * L, L), L)]
                bv = g * L + lanes
                c = chunk_of(iv)
                addr = c * L + lanes
                base = plsc.load_gather(cur, [addr])
                plsc.store_scatter(blist, [base], bv)
                plsc.store_scatter(vlist, [base], iv)
                plsc.addupdate_scatter(cur, [addr], ones)
                return carry

            lax.fori_loop(0, NGRP, append, 0, unroll=False)

            def extract_from(src_ref, gather_idx, c):
                s = pref[c]
                e = pref[c + 1]
                n_grp = lax.shift_right_logical(e - s + (L - 1), 4)

                def egroup(k, carry):
                    pos = s + k * L + lanes
                    m = pos < e
                    bv = plsc.load_gather(blist, [pos], mask=m)
                    vv = plsc.load_gather(vlist, [pos], mask=m)
                    for dd in range(HD):
                        ddv = dd + zeros16
                        vals = plsc.load_gather(
                            src_ref, gather_idx(ddv, vv), mask=m)
                        plsc.store_scatter(outs, [ddv, bv], vals, mask=m)
                    return carry

                lax.fori_loop(0, n_grp, egroup, 0, unroll=False)

            def extract(c, buf):
                bufv = buf + zeros16
                extract_from(
                    slab, lambda ddv, vv: [bufv, ddv, vv - c * VC], c)

            def ring(qq, carry):
                for b in range(NBUF):
                    c = qq * NBUF + b
                    pltpu.make_async_copy(
                        tab_hbm.at[pl.ds(row0, HD), pl.ds(0, VC)],
                        slab.at[b],
                        sems[b],
                    ).wait()
                    extract(c, b)

                    @pl.when(c + NBUF < NFULL)
                    def _():
                        fire(c + NBUF, b, sems[b], VC)

                    @pl.when(c + NBUF == NFULL)
                    def _():
                        fire(NFULL, b, sems[b], VLAST)

                return carry

            lax.fori_loop(0, NFULL // NBUF, ring, 0, unroll=False)

            pltpu.make_async_copy(
                tab_hbm.at[pl.ds(row0, HD), pl.ds(0, VLAST)],
                slab.at[0, :, pl.ds(0, VLAST)],
                sems[NFULL % NBUF],
            ).wait()
            extract(NFULL, 0)
            extract_from(
                tailslab, lambda ddv, vv: [ddv, vv - TAILV], NCH - 1)

            pltpu.async_copy(
                outs, out_hbm.at[f, pl.ds(pl.multiple_of(HD * h, 8), HD)],
                semf)

        def wait_flush(u):
            f = lax.rem(u, F)
            h = lax.div(u, F)
            pltpu.make_async_copy(
                out_hbm.at[f, pl.ds(pl.multiple_of(HD * h, 8), HD)],
                outs, semf).wait()

        for t in range(2):
            u = wid + NW * t

            @pl.when(u < UNITS)
            def _():
                if t > 0:
                    wait_flush(u - NW)
                unit_body(u)

        @pl.when(wid + NW < UNITS)
        def _():
            wait_flush(wid + NW)

        @pl.when(wid + NW >= UNITS)
        def _():
            wait_flush(wid)

    out_t = emb(idx_t, tab2, tail)
    return out_t.transpose(2, 0, 1)
